# Initial kernel scaffold; baseline (speedup 1.0000x reference)
import jax
import jax.numpy as jnp
from jax.experimental import pallas as pl


def kernel(x, edge_index, W1l, b1, W1r, W2l, b2, W2r):
    raise NotImplementedError("write your pallas kernel here")


# speedup vs baseline: 5.8509x; 5.8509x over previous
import jax
import jax.numpy as jnp
from jax import lax
from jax.experimental import pallas as pl
from jax.experimental.pallas import tpu as pltpu
from jax.experimental.pallas import tpu_sc as plsc

_NC, _NS = 2, 16
_NW = _NC * _NS
_K = 128
_ZR = 8


def _zero_fill(ref, rows, width):
  zeros16 = jnp.zeros((16,), jnp.float32)

  def zrow(i, _):
    for j in range(width // 16):
      ref[i, pl.ds(j * 16, 16)] = zeros16
    return 0

  lax.fori_loop(0, rows, zrow, 0, unroll=False)


def _make_spmm(N, E, D, with_gather):
  NB = E // _K
  assert E % _K == 0
  Np = -(-N // (8 * _NS)) * (8 * _NS)
  RPT = Np // _NS
  assert RPT % _ZR == 0

  mesh = plsc.VectorSubcoreMesh(core_axis_name="c", subcore_axis_name="s")
  out_type = [jax.ShapeDtypeStruct((_NC, Np, D), jnp.float32)]
  scratch = [
      pltpu.VMEM((_ZR, D), jnp.float32),
      pltpu.VMEM_SHARED((Np, D), jnp.float32),
      pltpu.VMEM((_K,), jnp.int32),
      pltpu.VMEM((_K,), jnp.int32),
      pltpu.VMEM((_K, D), jnp.float32),
      pltpu.SemaphoreType.DMA,
  ]

  def body(x_hbm, src_hbm, dst_hbm, out_hbm, zb, aggs, srcv, dstv, rowsv, sem):
    c = lax.axis_index("c")
    s = lax.axis_index("s")
    base = s * RPT
    _zero_fill(zb, _ZR, D)

    if not with_gather:
      ones16 = jnp.ones((16,), jnp.float32)

      def orow(i, _):
        for j in range(D // 16):
          rowsv[i, pl.ds(j * 16, 16)] = ones16
        return 0

      lax.fori_loop(0, _K, orow, 0, unroll=False)

    def zchunk(i, _):
      pltpu.sync_copy(zb, aggs.at[pl.ds(base + i * _ZR, _ZR)])
      return 0

    lax.fori_loop(0, RPT // _ZR, zchunk, 0, unroll=False)
    plsc.subcore_barrier()

    w = c * _NS + s
    nb = (NB - w + _NW - 1) // _NW

    def step(i, _):
      off = (w + i * _NW) * _K
      pltpu.sync_copy(dst_hbm.at[pl.ds(off, _K)], dstv)
      if with_gather:
        pltpu.sync_copy(src_hbm.at[pl.ds(off, _K)], srcv)
        pltpu.async_copy(x_hbm.at[srcv], rowsv, sem).wait()
      pltpu.sync_copy(rowsv, aggs.at[dstv], add=True)
      return 0

    lax.fori_loop(0, nb, step, 0, unroll=False)
    plsc.subcore_barrier()

    pltpu.sync_copy(aggs.at[pl.ds(base, RPT)], out_hbm.at[c, pl.ds(base, RPT)])

  return pl.kernel(body, out_type=out_type, mesh=mesh, scratch_types=scratch)


def _tc_linear(aggp, cntp, x, WlT, bl, WrT, relu):
  N, D = x.shape
  BR = 1000
  assert N % BR == 0

  def body(aggp_ref, cntp_ref, x_ref, wl_ref, bl_ref, wr_ref, o_ref):
    agg = aggp_ref[0] + aggp_ref[1]
    cnt = cntp_ref[0, :, 0:1] + cntp_ref[1, :, 0:1]
    m = agg * (1.0 / jnp.maximum(cnt, 1.0))
    h = (jnp.dot(m, wl_ref[...], preferred_element_type=jnp.float32)
         + bl_ref[...]
         + jnp.dot(x_ref[...], wr_ref[...], preferred_element_type=jnp.float32))
    if relu:
      h = jnp.maximum(h, 0.0)
    o_ref[...] = h

  return pl.pallas_call(
      body,
      grid=(N // BR,),
      in_specs=[
          pl.BlockSpec((2, BR, D), lambda i: (0, i, 0)),
          pl.BlockSpec((2, BR, D), lambda i: (0, i, 0)),
          pl.BlockSpec((BR, D), lambda i: (i, 0)),
          pl.BlockSpec((D, D), lambda i: (0, 0)),
          pl.BlockSpec((1, D), lambda i: (0, 0)),
          pl.BlockSpec((D, D), lambda i: (0, 0)),
      ],
      out_specs=pl.BlockSpec((BR, D), lambda i: (i, 0)),
      out_shape=jax.ShapeDtypeStruct((N, D), jnp.float32),
  )(aggp, cntp, x, WlT, bl, WrT)


def kernel(x, edge_index, W1l, b1, W1r, W2l, b2, W2r):
  N, D = x.shape
  E = edge_index.shape[1]
  src = edge_index[0]
  dst = edge_index[1]

  spmm = _make_spmm(N, E, D, with_gather=True)
  count = _make_spmm(N, E, D, with_gather=False)

  (cntp,) = count(x, src, dst)
  (agg1,) = spmm(x, src, dst)
  h = _tc_linear(agg1, cntp, x, W1l.T, b1.reshape(1, D), W1r.T, relu=True)
  (agg2,) = spmm(h, src, dst)
  out = _tc_linear(agg2, cntp, h, W2l.T, b2.reshape(1, D), W2r.T, relu=False)
  return out


# speedup vs baseline: 9.6121x; 1.6428x over previous
import jax
import jax.numpy as jnp
from jax import lax
from jax.experimental import pallas as pl
from jax.experimental.pallas import tpu as pltpu
from jax.experimental.pallas import tpu_sc as plsc

_NC, _NS = 2, 16
_NW = _NC * _NS
_K = 128
_CB = 8
_ZR = 8


def _zero_fill(ref, rows, width, value=0.0):
  v16 = jnp.full((16,), value, jnp.float32)

  def zrow(i, _):
    for j in range(width // 16):
      ref[i, pl.ds(j * 16, 16)] = v16
    return 0

  lax.fori_loop(0, rows, zrow, 0, unroll=False)


def _make_spmm(N, NB, D, with_gather):
  NPB = NB // _NW
  NCH = NPB // _CB
  assert NB % (_NW * _CB) == 0
  Np = -(-N // (8 * _NS)) * (8 * _NS)
  RPT = Np // _NS
  assert RPT % _ZR == 0

  mesh = plsc.VectorSubcoreMesh(core_axis_name="c", subcore_axis_name="s")
  out_type = [jax.ShapeDtypeStruct((_NC, Np, D), jnp.float32)]
  scratch = [
      pltpu.VMEM((_ZR, D), jnp.float32),
      pltpu.VMEM_SHARED((Np, D), jnp.float32),
      pltpu.VMEM((_CB, _K), jnp.int32),
      pltpu.VMEM((_CB, _K), jnp.int32),
      pltpu.VMEM((2, _K, D), jnp.float32),
      pltpu.SemaphoreType.DMA,
      pltpu.SemaphoreType.DMA,
  ]

  def body(x_hbm, src_hbm, dst_hbm, out_hbm, zb, aggs, srcv, dstv, rowsv,
           sem0, sem1):
    c = lax.axis_index("c")
    s = lax.axis_index("s")
    base = s * RPT
    sems = (sem0, sem1)
    _zero_fill(zb, _ZR, D)
    if not with_gather:
      _zero_fill(rowsv.at[0], _K, D, value=1.0)

    def zchunk(i, _):
      pltpu.sync_copy(zb, aggs.at[pl.ds(base + i * _ZR, _ZR)])
      return 0

    lax.fori_loop(0, RPT // _ZR, zchunk, 0, unroll=False)
    plsc.subcore_barrier()

    w = c * _NS + s

    def chunk(ic, _):
      row0 = w * NPB + ic * _CB
      pltpu.sync_copy(dst_hbm.at[pl.ds(row0, _CB)], dstv)
      if with_gather:
        pltpu.sync_copy(src_hbm.at[pl.ds(row0, _CB)], srcv)
        handles = [None, None]
        handles[0] = pltpu.async_copy(x_hbm.at[srcv.at[0]], rowsv.at[0],
                                      sems[0])
        for b in range(_CB):
          if b + 1 < _CB:
            nj = (b + 1) % 2
            handles[nj] = pltpu.async_copy(x_hbm.at[srcv.at[b + 1]],
                                           rowsv.at[nj], sems[nj])
          handles[b % 2].wait()
          pltpu.sync_copy(rowsv.at[b % 2], aggs.at[dstv.at[b]], add=True)
      else:
        for b in range(_CB):
          pltpu.sync_copy(rowsv.at[0], aggs.at[dstv.at[b]], add=True)
      return 0

    lax.fori_loop(0, NCH, chunk, 0, unroll=False)
    plsc.subcore_barrier()

    pltpu.sync_copy(aggs.at[pl.ds(base, RPT)], out_hbm.at[c, pl.ds(base, RPT)])

  return pl.kernel(body, out_type=out_type, mesh=mesh, scratch_types=scratch)


def _tc_linear(aggp, cntp, x, WlT, bl, WrT, relu):
  N, D = x.shape
  BR = 1000
  assert N % BR == 0

  def body(aggp_ref, cntp_ref, x_ref, wl_ref, bl_ref, wr_ref, o_ref):
    agg = aggp_ref[0] + aggp_ref[1]
    cnt = cntp_ref[0, :, 0:1] + cntp_ref[1, :, 0:1]
    m = agg * (1.0 / jnp.maximum(cnt, 1.0))
    h = (jnp.dot(m, wl_ref[...], preferred_element_type=jnp.float32)
         + bl_ref[...]
         + jnp.dot(x_ref[...], wr_ref[...], preferred_element_type=jnp.float32))
    if relu:
      h = jnp.maximum(h, 0.0)
    o_ref[...] = h

  return pl.pallas_call(
      body,
      grid=(N // BR,),
      in_specs=[
          pl.BlockSpec((2, BR, D), lambda i: (0, i, 0)),
          pl.BlockSpec((2, BR, D), lambda i: (0, i, 0)),
          pl.BlockSpec((BR, D), lambda i: (i, 0)),
          pl.BlockSpec((D, D), lambda i: (0, 0)),
          pl.BlockSpec((1, D), lambda i: (0, 0)),
          pl.BlockSpec((D, D), lambda i: (0, 0)),
      ],
      out_specs=pl.BlockSpec((BR, D), lambda i: (i, 0)),
      out_shape=jax.ShapeDtypeStruct((N, D), jnp.float32),
  )(aggp, cntp, x, WlT, bl, WrT)


def kernel(x, edge_index, W1l, b1, W1r, W2l, b2, W2r):
  N, D = x.shape
  E = edge_index.shape[1]

  Np = -(-N // (8 * _NS)) * (8 * _NS)
  unit = _NW * _CB * _K
  Ep = -(-E // unit) * unit
  NB = Ep // _K
  pad = Ep - E
  if pad:
    pad_src = (jnp.arange(pad, dtype=jnp.int32) * 97) % N
    pad_dst = N + (jnp.arange(pad, dtype=jnp.int32) % (Np - N))
    src = jnp.concatenate([edge_index[0], pad_src])
    dst = jnp.concatenate([edge_index[1], pad_dst])
  else:
    src = edge_index[0]
    dst = edge_index[1]
  src2d = src.reshape(NB, _K)
  dst2d = dst.reshape(NB, _K)

  spmm = _make_spmm(N, NB, D, with_gather=True)
  count = _make_spmm(N, NB, D, with_gather=False)

  (cntp,) = count(x, src2d, dst2d)
  (agg1,) = spmm(x, src2d, dst2d)
  h = _tc_linear(agg1, cntp, x, W1l.T, b1.reshape(1, D), W1r.T, relu=True)
  (agg2,) = spmm(h, src2d, dst2d)
  out = _tc_linear(agg2, cntp, h, W2l.T, b2.reshape(1, D), W2r.T, relu=False)
  return out


# speedup vs baseline: 10.8124x; 1.1249x over previous
import jax
import jax.numpy as jnp
from jax import lax
from jax.experimental import pallas as pl
from jax.experimental.pallas import tpu as pltpu
from jax.experimental.pallas import tpu_sc as plsc

_NC, _NS = 2, 16
_NW = _NC * _NS
_K = 128
_ZR = 8
_RING = 2
_GRP = 4
_IH = 2


def _zero_fill(ref, rows, width, value=0.0):
  v16 = jnp.full((16,), value, jnp.float32)

  def zrow(i, _):
    for j in range(width // 16):
      ref[i, pl.ds(j * 16, 16)] = v16
    return 0

  lax.fori_loop(0, rows, zrow, 0, unroll=False)


def _make_spmm(N, NB, D, with_gather):
  NPB = NB // _NW
  assert NB % _NW == 0 and NPB % _GRP == 0
  Np = -(-N // (8 * _NS)) * (8 * _NS)
  RPT = Np // _NS
  assert RPT % _ZR == 0

  mesh = plsc.VectorSubcoreMesh(core_axis_name="c", subcore_axis_name="s")
  out_type = [jax.ShapeDtypeStruct((_NC, Np, D), jnp.float32)]
  HPB = NPB // _IH
  assert HPB % _GRP == 0
  scratch = [
      pltpu.VMEM((_ZR, D), jnp.float32),
      pltpu.VMEM_SHARED((Np, D), jnp.float32),
      pltpu.VMEM((HPB, 2, _K), jnp.int32),
      pltpu.VMEM((_RING, _K, D), jnp.float32),
      pltpu.SemaphoreType.DMA,
      pltpu.SemaphoreType.DMA,
      pltpu.SemaphoreType.DMA,
  ]

  def body(x_hbm, idx_hbm, out_hbm, zb, aggs, idxv, rowsv,
           isem, gsem0, gsem1):
    c = lax.axis_index("c")
    s = lax.axis_index("s")
    base = s * RPT
    gsems = (gsem0, gsem1)
    w = c * _NS + s

    pltpu.async_copy(idx_hbm.at[pl.ds(w * NPB, HPB)], idxv, isem)

    _zero_fill(zb, _ZR, D)
    if not with_gather:
      _zero_fill(rowsv.at[0], _K, D, value=1.0)

    def zchunk(i, _):
      pltpu.sync_copy(zb, aggs.at[pl.ds(base + i * _ZR, _ZR)])
      return 0

    lax.fori_loop(0, RPT // _ZR, zchunk, 0, unroll=False)

    pltpu.make_async_copy(idx_hbm.at[pl.ds(0, HPB)], idxv, isem).wait()
    plsc.subcore_barrier()

    for h in range(_IH):
      if h > 0:
        pltpu.sync_copy(idx_hbm.at[pl.ds(w * NPB + h * HPB, HPB)], idxv)
      if with_gather:
        for p in range(_RING - 1):
          pltpu.async_copy(x_hbm.at[idxv.at[p, 0]], rowsv.at[p], gsems[p])

        def grp(g, _):
          b0 = g * _GRP
          for u in range(_GRP):
            b = b0 + u
            nxt = (u + _RING - 1) % _RING

            @pl.when(b + _RING - 1 < HPB)
            def _():
              pltpu.async_copy(x_hbm.at[idxv.at[b + _RING - 1, 0]],
                               rowsv.at[nxt], gsems[nxt])

            pltpu.make_async_copy(x_hbm.at[pl.ds(0, _K)], rowsv.at[u % _RING],
                                  gsems[u % _RING]).wait()
            pltpu.sync_copy(rowsv.at[u % _RING], aggs.at[idxv.at[b, 1]],
                            add=True)
          return 0

        lax.fori_loop(0, HPB // _GRP, grp, 0, unroll=False)
      else:

        def blk(b, _):
          pltpu.sync_copy(rowsv.at[0], aggs.at[idxv.at[b, 1]], add=True)
          return 0

        lax.fori_loop(0, HPB, blk, 0, unroll=False)

    plsc.subcore_barrier()

    pltpu.sync_copy(aggs.at[pl.ds(base, RPT)], out_hbm.at[c, pl.ds(base, RPT)])

  return pl.kernel(body, out_type=out_type, mesh=mesh, scratch_types=scratch)


def _tc_linear(aggp, cntp, x, WlT, bl, WrT, relu):
  N, D = x.shape
  BR = 1000
  assert N % BR == 0

  def body(aggp_ref, cntp_ref, x_ref, wl_ref, bl_ref, wr_ref, o_ref):
    agg = aggp_ref[0] + aggp_ref[1]
    cnt = cntp_ref[0, :, 0:1] + cntp_ref[1, :, 0:1]
    m = agg * (1.0 / jnp.maximum(cnt, 1.0))
    h = (jnp.dot(m, wl_ref[...], preferred_element_type=jnp.float32)
         + bl_ref[...]
         + jnp.dot(x_ref[...], wr_ref[...], preferred_element_type=jnp.float32))
    if relu:
      h = jnp.maximum(h, 0.0)
    o_ref[...] = h

  return pl.pallas_call(
      body,
      grid=(N // BR,),
      in_specs=[
          pl.BlockSpec((2, BR, D), lambda i: (0, i, 0)),
          pl.BlockSpec((2, BR, D), lambda i: (0, i, 0)),
          pl.BlockSpec((BR, D), lambda i: (i, 0)),
          pl.BlockSpec((D, D), lambda i: (0, 0)),
          pl.BlockSpec((1, D), lambda i: (0, 0)),
          pl.BlockSpec((D, D), lambda i: (0, 0)),
      ],
      out_specs=pl.BlockSpec((BR, D), lambda i: (i, 0)),
      out_shape=jax.ShapeDtypeStruct((N, D), jnp.float32),
  )(aggp, cntp, x, WlT, bl, WrT)


def kernel(x, edge_index, W1l, b1, W1r, W2l, b2, W2r):
  N, D = x.shape
  E = edge_index.shape[1]

  Np = -(-N // (8 * _NS)) * (8 * _NS)
  unit = _NW * 8 * _K
  Ep = -(-E // unit) * unit
  NB = Ep // _K
  pad = Ep - E
  if pad:
    pad_src = (jnp.arange(pad, dtype=jnp.int32) * 97) % N
    pad_dst = N + (jnp.arange(pad, dtype=jnp.int32) % (Np - N))
    src = jnp.concatenate([edge_index[0], pad_src])
    dst = jnp.concatenate([edge_index[1], pad_dst])
  else:
    src = edge_index[0]
    dst = edge_index[1]
  idx = jnp.stack([src.reshape(NB, _K), dst.reshape(NB, _K)], axis=1)

  spmm = _make_spmm(N, NB, D, with_gather=True)
  count = _make_spmm(N, NB, D, with_gather=False)

  (cntp,) = count(x, idx)
  (agg1,) = spmm(x, idx)
  h = _tc_linear(agg1, cntp, x, W1l.T, b1.reshape(1, D), W1r.T, relu=True)
  (agg2,) = spmm(h, idx)
  out = _tc_linear(agg2, cntp, h, W2l.T, b2.reshape(1, D), W2r.T, relu=False)
  return out


# speedup vs baseline: 10.9194x; 1.0099x over previous
import jax
import jax.numpy as jnp
from jax import lax
from jax.experimental import pallas as pl
from jax.experimental.pallas import tpu as pltpu
from jax.experimental.pallas import tpu_sc as plsc

_NC, _NS = 2, 16
_NW = _NC * _NS
_K = 128
_ZR = 8
_RING = 2
_GRP = 4
_IH = 2


def _zero_fill(ref, rows, width, value=0.0):
  v16 = jnp.full((16,), value, jnp.float32)

  def zrow(i, _):
    for j in range(width // 16):
      ref[i, pl.ds(j * 16, 16)] = v16
    return 0

  lax.fori_loop(0, rows, zrow, 0, unroll=False)


def _make_spmm(N, NB, D, with_count):
  NPB = NB // _NW
  assert NB % _NW == 0 and NPB % _GRP == 0
  Np = -(-N // (8 * _NS)) * (8 * _NS)
  RPT = Np // _NS
  assert RPT % _ZR == 0

  mesh = plsc.VectorSubcoreMesh(core_axis_name="c", subcore_axis_name="s")
  out_type = [jax.ShapeDtypeStruct((_NC, Np, D), jnp.float32)]
  if with_count:
    out_type.append(jax.ShapeDtypeStruct((_NC, Np, D), jnp.float32))
  HPB = NPB // _IH
  assert HPB % _GRP == 0
  scratch = [
      pltpu.VMEM((_ZR, D), jnp.float32),
      pltpu.VMEM_SHARED((Np, D), jnp.float32),
      pltpu.VMEM((HPB, 2, _K), jnp.int32),
      pltpu.VMEM((_RING, _K, D), jnp.float32),
      pltpu.SemaphoreType.DMA,
      pltpu.SemaphoreType.DMA,
      pltpu.SemaphoreType.DMA,
  ]

  def body(x_hbm, idx_hbm, *rest):
    if with_count:
      out_hbm, cnt_hbm, zb, aggs, idxv, rowsv, isem, gsem0, gsem1 = rest
    else:
      out_hbm, zb, aggs, idxv, rowsv, isem, gsem0, gsem1 = rest
      cnt_hbm = None
    c = lax.axis_index("c")
    s = lax.axis_index("s")
    base = s * RPT
    gsems = (gsem0, gsem1)
    w = c * _NS + s

    pltpu.async_copy(idx_hbm.at[pl.ds(w * NPB, HPB)], idxv, isem)

    _zero_fill(zb, _ZR, D)
    if with_count:
      _zero_fill(rowsv.at[1], _K, D, value=1.0)

    def zchunk(i, _):
      pltpu.sync_copy(zb, aggs.at[pl.ds(base + i * _ZR, _ZR)])
      return 0

    lax.fori_loop(0, RPT // _ZR, zchunk, 0, unroll=False)

    pltpu.make_async_copy(idx_hbm.at[pl.ds(0, HPB)], idxv, isem).wait()
    plsc.subcore_barrier()

    if with_count:
      for h in range(_IH):
        if h > 0:
          pltpu.sync_copy(idx_hbm.at[pl.ds(w * NPB + h * HPB, HPB)], idxv)

        def blk(b, _):
          pltpu.sync_copy(rowsv.at[1], aggs.at[idxv.at[b, 1]], add=True)
          return 0

        lax.fori_loop(0, HPB, blk, 0, unroll=False)

      plsc.subcore_barrier()
      pltpu.sync_copy(aggs.at[pl.ds(base, RPT)],
                      cnt_hbm.at[c, pl.ds(base, RPT)])

      lax.fori_loop(0, RPT // _ZR, zchunk, 0, unroll=False)
      plsc.subcore_barrier()

    for h in range(_IH):
      if h > 0 or with_count:
        pltpu.sync_copy(idx_hbm.at[pl.ds(w * NPB + h * HPB, HPB)], idxv)
      for p in range(_RING - 1):
        pltpu.async_copy(x_hbm.at[idxv.at[p, 0]], rowsv.at[p], gsems[p])

      def grp(g, _):
        b0 = g * _GRP
        for u in range(_GRP):
          b = b0 + u
          nxt = (u + _RING - 1) % _RING

          @pl.when(b + _RING - 1 < HPB)
          def _():
            pltpu.async_copy(x_hbm.at[idxv.at[b + _RING - 1, 0]],
                             rowsv.at[nxt], gsems[nxt])

          pltpu.make_async_copy(x_hbm.at[pl.ds(0, _K)], rowsv.at[u % _RING],
                                gsems[u % _RING]).wait()
          pltpu.sync_copy(rowsv.at[u % _RING], aggs.at[idxv.at[b, 1]],
                          add=True)
        return 0

      lax.fori_loop(0, HPB // _GRP, grp, 0, unroll=False)

    plsc.subcore_barrier()

    pltpu.sync_copy(aggs.at[pl.ds(base, RPT)], out_hbm.at[c, pl.ds(base, RPT)])

  return pl.kernel(body, out_type=out_type, mesh=mesh, scratch_types=scratch)


def _tc_linear(aggp, cntp, x, WlT, bl, WrT, relu):
  N, D = x.shape
  BR = 1000
  assert N % BR == 0

  def body(aggp_ref, cntp_ref, x_ref, wl_ref, bl_ref, wr_ref, o_ref):
    agg = aggp_ref[0] + aggp_ref[1]
    cnt = cntp_ref[0, :, 0:1] + cntp_ref[1, :, 0:1]
    m = agg * (1.0 / jnp.maximum(cnt, 1.0))
    h = (jnp.dot(m, wl_ref[...], preferred_element_type=jnp.float32)
         + bl_ref[...]
         + jnp.dot(x_ref[...], wr_ref[...], preferred_element_type=jnp.float32))
    if relu:
      h = jnp.maximum(h, 0.0)
    o_ref[...] = h

  return pl.pallas_call(
      body,
      grid=(N // BR,),
      in_specs=[
          pl.BlockSpec((2, BR, D), lambda i: (0, i, 0)),
          pl.BlockSpec((2, BR, D), lambda i: (0, i, 0)),
          pl.BlockSpec((BR, D), lambda i: (i, 0)),
          pl.BlockSpec((D, D), lambda i: (0, 0)),
          pl.BlockSpec((1, D), lambda i: (0, 0)),
          pl.BlockSpec((D, D), lambda i: (0, 0)),
      ],
      out_specs=pl.BlockSpec((BR, D), lambda i: (i, 0)),
      out_shape=jax.ShapeDtypeStruct((N, D), jnp.float32),
  )(aggp, cntp, x, WlT, bl, WrT)


def kernel(x, edge_index, W1l, b1, W1r, W2l, b2, W2r):
  N, D = x.shape
  E = edge_index.shape[1]

  Np = -(-N // (8 * _NS)) * (8 * _NS)
  unit = _NW * 8 * _K
  Ep = -(-E // unit) * unit
  NB = Ep // _K
  pad = Ep - E
  if pad:
    pad_src = (jnp.arange(pad, dtype=jnp.int32) * 97) % N
    pad_dst = N + (jnp.arange(pad, dtype=jnp.int32) % (Np - N))
    src = jnp.concatenate([edge_index[0], pad_src])
    dst = jnp.concatenate([edge_index[1], pad_dst])
  else:
    src = edge_index[0]
    dst = edge_index[1]
  idx = jnp.stack([src.reshape(NB, _K), dst.reshape(NB, _K)], axis=1)

  spmm_count = _make_spmm(N, NB, D, with_count=True)
  spmm = _make_spmm(N, NB, D, with_count=False)

  agg1, cntp = spmm_count(x, idx)
  h = _tc_linear(agg1, cntp, x, W1l.T, b1.reshape(1, D), W1r.T, relu=True)
  (agg2,) = spmm(h, idx)
  out = _tc_linear(agg2, cntp, h, W2l.T, b2.reshape(1, D), W2r.T, relu=False)
  return out
